# Initial kernel scaffold; baseline (speedup 1.0000x reference)
import jax
import jax.numpy as jnp
from jax.experimental import pallas as pl


def kernel(input_vec, table, bias):
    raise NotImplementedError("write your pallas kernel here")


# speedup vs baseline: 2.2870x; 2.2870x over previous
import functools

import jax
import jax.numpy as jnp
from jax import lax
from jax.experimental import pallas as pl
from jax.experimental.pallas import tpu as pltpu
from jax.experimental.pallas import tpu_sc as plsc

VOCAB = 100
PAD = 128
EMB = 10
REP = 8
N_TOKENS = 4096 * 50
IDXW = 128
N_IDX_ROWS = N_TOKENS // IDXW

_info = plsc.get_sparse_core_info()
NC, NS = _info.num_cores, _info.num_subcores
NW = NC * NS
ROWS_PER_W = N_IDX_ROWS // NW


def _gram_body(t_ref, b_ref, g_ref):
    t = t_ref[...]
    g = jnp.dot(t, t.T, preferred_element_type=jnp.float32) + b_ref[...]
    g = jnp.pad(g, ((0, 0), (0, PAD - VOCAB)))
    g_ref[...] = jnp.concatenate([g] * REP, axis=0)


def _compute_gram(table, bias):
    return pl.pallas_call(
        _gram_body,
        out_shape=jax.ShapeDtypeStruct((REP * VOCAB, PAD), jnp.float32),
    )(table, bias.reshape(1, VOCAB))


_sc_mesh = plsc.VectorSubcoreMesh(core_axis_name="c", subcore_axis_name="s")


@functools.partial(
    pl.kernel,
    out_type=jax.ShapeDtypeStruct((N_TOKENS, VOCAB), jnp.float32),
    mesh=_sc_mesh,
    scratch_types=[
        pltpu.VMEM((ROWS_PER_W, 1, IDXW), jnp.int32),
        pltpu.VMEM((IDXW, PAD), jnp.float32),
        pltpu.VMEM((IDXW, VOCAB), jnp.float32),
        pltpu.SemaphoreType.DMA,
    ],
)
def _sc_gather(idx_hbm, g_hbm, out_hbm, idx_v, rows_v, outb_v, sem):
    sid = lax.axis_index("s")
    wid = sid * NC + lax.axis_index("c")
    base = wid * ROWS_PER_W

    pltpu.sync_copy(idx_hbm.at[wid], idx_v)

    rep_off = (wid % REP) * VOCAB

    def _adjust(i, _):
        def _adj16(k, _):
            v = idx_v[i, 0, pl.ds(k * 16, 16)]
            idx_v[i, 0, pl.ds(k * 16, 16)] = v + rep_off
            return 0
        return lax.fori_loop(0, IDXW // 16, _adj16, 0)

    lax.fori_loop(0, ROWS_PER_W, _adjust, 0)

    def _compact(r, _):
        for c in range(VOCAB // 16):
            outb_v[r, pl.ds(c * 16, 16)] = rows_v[r, pl.ds(c * 16, 16)]
        outb_v[r, pl.ds(VOCAB - 16, 16)] = rows_v[r, pl.ds(VOCAB - 16, 16)]
        return 0

    for i in range(ROWS_PER_W):
        pltpu.async_copy(g_hbm.at[idx_v.at[i, 0]], rows_v, sem).wait()
        lax.fori_loop(0, IDXW, _compact, 0)
        pltpu.sync_copy(outb_v, out_hbm.at[pl.ds((base + i) * IDXW, IDXW)])


def kernel(input_vec, table, bias):
    idx = input_vec.reshape(NW, ROWS_PER_W, 1, IDXW).astype(jnp.int32)
    g = _compute_gram(table, bias)
    out = _sc_gather(idx, g)
    return out.reshape(input_vec.shape[0], input_vec.shape[1], VOCAB)


# speedup vs baseline: 2.7536x; 1.2040x over previous
import functools

import jax
import jax.numpy as jnp
from jax import lax
from jax.experimental import pallas as pl
from jax.experimental.pallas import tpu as pltpu
from jax.experimental.pallas import tpu_sc as plsc

VOCAB = 100
PAD = 128
EMB = 10
REP = 8
B = 4096
SEQ = 50

_info = plsc.get_sparse_core_info()
NC, NS = _info.num_cores, _info.num_subcores
NW = NC * NS
B_PER_W = B // NW
UNITS = B_PER_W // 2


def _gram_body(t_ref, b_ref, g_ref):
    t = t_ref[...]
    g = jnp.dot(t, t.T, preferred_element_type=jnp.float32) + b_ref[...]
    g = jnp.pad(g, ((0, 0), (0, PAD - VOCAB)))
    g_ref[...] = jnp.concatenate([g] * REP, axis=0)


def _compute_gram(table, bias):
    return pl.pallas_call(
        _gram_body,
        out_shape=jax.ShapeDtypeStruct((REP * VOCAB, PAD), jnp.float32),
    )(table, bias.reshape(1, VOCAB))


_sc_mesh = plsc.VectorSubcoreMesh(core_axis_name="c", subcore_axis_name="s")


@functools.partial(
    pl.kernel,
    out_type=jax.ShapeDtypeStruct((B, SEQ, VOCAB), jnp.float32),
    mesh=_sc_mesh,
    scratch_types=[
        pltpu.VMEM((B_PER_W, SEQ), jnp.int32),
        pltpu.VMEM((UNITS, 1, 2 * SEQ), jnp.int32),
        pltpu.VMEM((2 * SEQ, PAD), jnp.float32),
        pltpu.VMEM((2, SEQ, VOCAB), jnp.float32),
        pltpu.SemaphoreType.DMA,
    ],
)
def _sc_gather(idx_hbm, g_hbm, out_hbm, idx_v, idxf_v, rows_v, outb_v, sem):
    sid = lax.axis_index("s")
    wid = sid * NC + lax.axis_index("c")
    b0 = wid * B_PER_W

    pltpu.sync_copy(idx_hbm.at[pl.ds(b0, B_PER_W)], idx_v)

    rep_off = (wid % REP) * VOCAB

    def _flatten(u, _):
        for h in range(2):
            src = 2 * u + h
            for (so, do) in ((0, 0), (16, 16), (32, 32), (34, 34)):
                v = idx_v[src, pl.ds(so, 16)]
                idxf_v[u, 0, pl.ds(h * SEQ + do, 16)] = v + rep_off
        return 0

    lax.fori_loop(0, UNITS, _flatten, 0)

    def _compact(l, _):
        for h in range(2):
            for c in range(VOCAB // 16):
                outb_v[h, l, pl.ds(c * 16, 16)] = rows_v[h * SEQ + l, pl.ds(c * 16, 16)]
            outb_v[h, l, pl.ds(VOCAB - 16, 16)] = rows_v[h * SEQ + l, pl.ds(VOCAB - 16, 16)]
        return 0

    for u in range(UNITS):
        pltpu.async_copy(g_hbm.at[idxf_v.at[u, 0]], rows_v, sem).wait()
        lax.fori_loop(0, SEQ, _compact, 0)
        pltpu.sync_copy(outb_v, out_hbm.at[pl.ds(b0 + 2 * u, 2)])


def kernel(input_vec, table, bias):
    g = _compute_gram(table, bias)
    return _sc_gather(input_vec.astype(jnp.int32), g)


# speedup vs baseline: 3.2392x; 1.1763x over previous
import functools

import jax
import jax.numpy as jnp
from jax import lax
from jax.experimental import pallas as pl
from jax.experimental.pallas import tpu as pltpu
from jax.experimental.pallas import tpu_sc as plsc

VOCAB = 100
PAD = 128
EMB = 10
REP = 8
B = 4096
SEQ = 50

_info = plsc.get_sparse_core_info()
NC, NS = _info.num_cores, _info.num_subcores
NW = NC * NS
B_PER_W = B // NW
UNITS = B_PER_W // 2


def _gram_body(t_ref, b_ref, g_ref):
    t = t_ref[...]
    g = jnp.dot(t, t.T, preferred_element_type=jnp.float32) + b_ref[...]
    g = jnp.pad(g, ((0, 0), (0, PAD - VOCAB)))
    g_ref[...] = jnp.concatenate([g] * REP, axis=0)


def _compute_gram(table, bias):
    return pl.pallas_call(
        _gram_body,
        out_shape=jax.ShapeDtypeStruct((REP * VOCAB, PAD), jnp.float32),
    )(table, bias.reshape(1, VOCAB))


_sc_mesh = plsc.VectorSubcoreMesh(core_axis_name="c", subcore_axis_name="s")


@functools.partial(
    pl.kernel,
    out_type=jax.ShapeDtypeStruct((B, SEQ, VOCAB), jnp.float32),
    mesh=_sc_mesh,
    scratch_types=[
        pltpu.VMEM((B_PER_W, SEQ), jnp.int32),
        pltpu.VMEM((UNITS, 1, 2 * SEQ), jnp.int32),
        pltpu.VMEM((4, 2 * SEQ, PAD), jnp.float32),
        pltpu.VMEM((2, 2, SEQ, VOCAB), jnp.float32),
        pltpu.SemaphoreType.DMA,
        pltpu.SemaphoreType.DMA,
        pltpu.SemaphoreType.DMA,
        pltpu.SemaphoreType.DMA,
        pltpu.SemaphoreType.DMA,
        pltpu.SemaphoreType.DMA,
    ],
)
def _sc_gather(idx_hbm, g_hbm, out_hbm, idx_v, idxf_v, rows_v, outb_v,
               gsem0, gsem1, gsem2, gsem3, wsem0, wsem1):
    sid = lax.axis_index("s")
    wid = sid * NC + lax.axis_index("c")
    b0 = wid * B_PER_W
    gsems = (gsem0, gsem1, gsem2, gsem3)
    wsems = (wsem0, wsem1)

    pltpu.sync_copy(idx_hbm.at[pl.ds(b0, B_PER_W)], idx_v)

    rep_off = (wid % REP) * VOCAB

    def _flatten(u, _):
        for h in range(2):
            src = 2 * u + h
            for (so, do) in ((0, 0), (16, 16), (32, 32), (34, 34)):
                v = idx_v[src, pl.ds(so, 16)]
                idxf_v[u, 0, pl.ds(h * SEQ + do, 16)] = v + rep_off
        return 0

    lax.fori_loop(0, UNITS, _flatten, 0)

    def _start_gather(u):
        p = u % 4
        return pltpu.async_copy(
            g_hbm.at[idxf_v.at[u, 0]], rows_v.at[p], gsems[p])

    def _compact_unit(u):
        p = u % 4
        q = u % 2

        def _compact(i, _):
            for j in range(2):
                l = 2 * i + j
                for h in range(2):
                    for c in range(VOCAB // 16):
                        outb_v[q, h, l, pl.ds(c * 16, 16)] = (
                            rows_v[p, h * SEQ + l, pl.ds(c * 16, 16)])
                    outb_v[q, h, l, pl.ds(VOCAB - 16, 16)] = (
                        rows_v[p, h * SEQ + l, pl.ds(VOCAB - 16, 16)])
            return 0

        lax.fori_loop(0, SEQ // 2, _compact, 0)

    gathers = {}
    writes = {}
    for u in range(3):
        gathers[u] = _start_gather(u)
    for u in range(UNITS):
        gathers.pop(u).wait()
        if u + 3 < UNITS:
            gathers[u + 3] = _start_gather(u + 3)
        if u - 2 in writes:
            writes.pop(u - 2).wait()
        _compact_unit(u)
        q = u % 2
        writes[u] = pltpu.async_copy(
            outb_v.at[q], out_hbm.at[pl.ds(b0 + 2 * u, 2)], wsems[q])
    for u in sorted(writes):
        writes.pop(u).wait()


def kernel(input_vec, table, bias):
    g = _compute_gram(table, bias)
    return _sc_gather(input_vec.astype(jnp.int32), g)
